# Initial kernel scaffold; baseline (speedup 1.0000x reference)
import jax
import jax.numpy as jnp
from jax.experimental import pallas as pl


def kernel(x, edge_index, W1, b1, W2, b2):
    raise NotImplementedError("write your pallas kernel here")


# speedup vs baseline: 5.6795x; 5.6795x over previous
import functools

import jax
import jax.numpy as jnp
import numpy as np
from jax import lax
from jax.experimental import pallas as pl
from jax.experimental.pallas import tpu as pltpu
from jax.experimental.pallas import tpu_sc as plsc

N = 10000
E = 320000
D = 128
NPAD = 10240
HALF = NPAD // 2
RPT = HALF // 16
EPAD = 327680
CHUNK = 1024
NCHUNK = EPAD // (16 * CHUNK)

_mesh = plsc.VectorSubcoreMesh(core_axis_name="c", subcore_axis_name="s")

_ONES = np.ones((128, D), np.float32)


@functools.partial(
    pl.kernel,
    out_type=jax.ShapeDtypeStruct((2 * NPAD, D), jnp.float32),
    mesh=_mesh,
    scratch_types=[
        pltpu.VMEM((8, 128), jnp.int32),
        pltpu.VMEM((128, D), jnp.float32),
        pltpu.VMEM((16, D), jnp.float32),
        pltpu.VMEM_SHARED((NPAD, D), jnp.float32),
    ],
)
def _deg_kernel(col2d_hbm, ones_hbm, deg_out, cbuf, ones_v, zbuf, deg_sh):
    c = lax.axis_index("c")
    s = lax.axis_index("s")
    t0 = s * (NPAD // 16)
    pltpu.sync_copy(ones_hbm, ones_v)
    zero16 = jnp.zeros((16,), jnp.float32)
    for r in range(16):
        for g in range(D // 16):
            zbuf[r, pl.ds(g * 16, 16)] = zero16
    for r in range(NPAD // 16 // 16):
        pltpu.sync_copy(zbuf, deg_sh.at[pl.ds(t0 + r * 16, 16)])
    plsc.subcore_barrier()

    rowbase0 = c * (EPAD // 2 // 128) + s * (EPAD // 32 // 128)

    def chunk(j, carry):
        cb = pl.multiple_of(rowbase0 + j * 8, 8)
        pltpu.sync_copy(col2d_hbm.at[pl.ds(cb, 8)], cbuf)
        for k in range(8):
            pltpu.sync_copy(ones_v, deg_sh.at[cbuf.at[k]], add=True)
        return carry

    lax.fori_loop(0, EPAD // 32 // CHUNK, chunk, 0)
    plsc.subcore_barrier()
    pltpu.sync_copy(deg_sh.at[pl.ds(t0, NPAD // 16)],
                    deg_out.at[pl.ds(c * NPAD + t0, NPAD // 16)])


@functools.partial(
    pl.kernel,
    out_type=jax.ShapeDtypeStruct((NPAD, D), jnp.float32),
    mesh=_mesh,
    scratch_types=[
        pltpu.VMEM((CHUNK,), jnp.int32),
        pltpu.VMEM((8, 128), jnp.int32),
        pltpu.VMEM((8, 128), jnp.int32),
        pltpu.VMEM((CHUNK // 2, D), jnp.float32),
        pltpu.VMEM_SHARED((HALF + 16, D), jnp.float32),
        pltpu.SemaphoreType.DMA,
    ],
)
def _prop_kernel(hs_hbm, rows_hbm, col2d_hbm, acc_out,
                 rbuf, cbuf, lbuf, gbuf, acc_sh, sem):
    c = lax.axis_index("c")
    s = lax.axis_index("s")
    t0 = s * RPT
    pltpu.sync_copy(hs_hbm.at[pl.ds(c * HALF + t0, RPT)],
                    acc_sh.at[pl.ds(t0, RPT)])
    plsc.subcore_barrier()

    lo = c * HALF
    dummy = HALF + s
    ebase = s * (EPAD // 16)

    def chunk(j, carry):
        b = ebase + j * CHUNK
        pltpu.sync_copy(rows_hbm.at[pl.ds(b, CHUNK)], rbuf)
        cb = pl.multiple_of(b // 128, 8)
        pltpu.sync_copy(col2d_hbm.at[pl.ds(cb, 8)], cbuf)
        for k in range(8):
            for g in range(D // 16):
                v = cbuf[k, pl.ds(g * 16, 16)]
                lc = v - lo
                ok = (lc >= 0) & (lc < HALF)
                lbuf[k, pl.ds(g * 16, 16)] = jnp.where(ok, lc, dummy)
        for hf in range(2):
            pltpu.async_copy(
                hs_hbm.at[rbuf.at[pl.ds(hf * (CHUNK // 2), CHUNK // 2)]],
                gbuf, sem).wait()
            for k in range(4):
                pltpu.sync_copy(gbuf.at[pl.ds(k * 128, 128)],
                                acc_sh.at[lbuf.at[hf * 4 + k]], add=True)
        return carry

    lax.fori_loop(0, NCHUNK, chunk, 0)
    plsc.subcore_barrier()
    pltpu.sync_copy(acc_sh.at[pl.ds(t0, RPT)],
                    acc_out.at[pl.ds(c * HALF + t0, RPT)])


def _pre_body(x_ref, w_ref, deg_ref, out_ref):
    s = lax.rsqrt(deg_ref[...])[:, None]
    h = jnp.dot(x_ref[...], w_ref[...].T, preferred_element_type=jnp.float32)
    out_ref[...] = h * s


def _pre_call(x, W1, deg):
    return pl.pallas_call(
        _pre_body,
        grid=(NPAD // 1024,),
        in_specs=[
            pl.BlockSpec((1024, D), lambda i: (i, 0)),
            pl.BlockSpec((D, D), lambda i: (0, 0)),
            pl.BlockSpec((1024,), lambda i: (i,)),
        ],
        out_specs=pl.BlockSpec((1024, D), lambda i: (i, 0)),
        out_shape=jax.ShapeDtypeStruct((NPAD, D), jnp.float32),
    )(x, W1, deg)


def _mid_body(acc_ref, deg_ref, b_ref, w_ref, out_ref):
    s = lax.rsqrt(deg_ref[...])[:, None]
    x1 = acc_ref[...] * s + b_ref[...][None, :]
    xr = jnp.maximum(x1, 0.0)
    h2 = jnp.dot(xr, w_ref[...].T, preferred_element_type=jnp.float32)
    out_ref[...] = h2 * s


def _mid_call(acc, deg, b1, W2):
    return pl.pallas_call(
        _mid_body,
        grid=(NPAD // 1024,),
        in_specs=[
            pl.BlockSpec((1024, D), lambda i: (i, 0)),
            pl.BlockSpec((1024,), lambda i: (i,)),
            pl.BlockSpec((D,), lambda i: (0,)),
            pl.BlockSpec((D, D), lambda i: (0, 0)),
        ],
        out_specs=pl.BlockSpec((1024, D), lambda i: (i, 0)),
        out_shape=jax.ShapeDtypeStruct((NPAD, D), jnp.float32),
    )(acc, deg, b1, W2)


def _final_body(acc_ref, deg_ref, b_ref, out_ref):
    s = lax.rsqrt(deg_ref[...])[:, None]
    out_ref[...] = acc_ref[...] * s + b_ref[...][None, :]


def _final_call(acc, deg, b2):
    return pl.pallas_call(
        _final_body,
        grid=(NPAD // 1024,),
        in_specs=[
            pl.BlockSpec((1024, D), lambda i: (i, 0)),
            pl.BlockSpec((1024,), lambda i: (i,)),
            pl.BlockSpec((D,), lambda i: (0,)),
        ],
        out_specs=pl.BlockSpec((1024, D), lambda i: (i, 0)),
        out_shape=jax.ShapeDtypeStruct((NPAD, D), jnp.float32),
    )(acc, deg, b2)


def kernel(x, edge_index, W1, b1, W2, b2):
    ei = edge_index.astype(jnp.int32)
    row, col = ei[0], ei[1]
    pad = EPAD - E
    row_p = jnp.concatenate([row, jnp.zeros((pad,), jnp.int32)])
    col_p = jnp.concatenate([col, jnp.full((pad,), N, jnp.int32)])
    col2d = col_p.reshape(EPAD // 128, 128)

    deg2 = _deg_kernel(col2d, jnp.asarray(_ONES))
    deg = deg2[:NPAD, 0] + deg2[NPAD:, 0] + 1.0

    hs1 = _pre_call(x, W1, deg)
    acc1 = _prop_kernel(hs1, row_p, col2d)
    hs2 = _mid_call(acc1, deg, b1, W2)
    acc2 = _prop_kernel(hs2, row_p, col2d)
    return _final_call(acc2, deg, b2)[:N]


# speedup vs baseline: 5.8517x; 1.0303x over previous
import functools

import jax
import jax.numpy as jnp
import numpy as np
from jax import lax
from jax.experimental import pallas as pl
from jax.experimental.pallas import tpu as pltpu
from jax.experimental.pallas import tpu_sc as plsc

N = 10000
E = 320000
D = 128
NPAD = 10240
HALF = NPAD // 2
RPT = HALF // 16
EPAD = 327680
CHUNK = 1024
NCHUNK = EPAD // (16 * CHUNK)

_mesh = plsc.VectorSubcoreMesh(core_axis_name="c", subcore_axis_name="s")

_ONES = np.ones((128, D), np.float32)


@functools.partial(
    pl.kernel,
    out_type=jax.ShapeDtypeStruct((2 * NPAD, D), jnp.float32),
    mesh=_mesh,
    scratch_types=[
        pltpu.VMEM((8, 128), jnp.int32),
        pltpu.VMEM((128, D), jnp.float32),
        pltpu.VMEM((16, D), jnp.float32),
        pltpu.VMEM_SHARED((NPAD, D), jnp.float32),
    ],
)
def _deg_kernel(col2d_hbm, ones_hbm, deg_out, cbuf, ones_v, zbuf, deg_sh):
    c = lax.axis_index("c")
    s = lax.axis_index("s")
    t0 = s * (NPAD // 16)
    pltpu.sync_copy(ones_hbm, ones_v)
    zero16 = jnp.zeros((16,), jnp.float32)
    for r in range(16):
        for g in range(D // 16):
            zbuf[r, pl.ds(g * 16, 16)] = zero16
    for r in range(NPAD // 16 // 16):
        pltpu.sync_copy(zbuf, deg_sh.at[pl.ds(t0 + r * 16, 16)])
    plsc.subcore_barrier()

    rowbase0 = c * (EPAD // 2 // 128) + s * (EPAD // 32 // 128)

    def chunk(j, carry):
        cb = pl.multiple_of(rowbase0 + j * 8, 8)
        pltpu.sync_copy(col2d_hbm.at[pl.ds(cb, 8)], cbuf)
        for k in range(8):
            pltpu.sync_copy(ones_v, deg_sh.at[cbuf.at[k]], add=True)
        return carry

    lax.fori_loop(0, EPAD // 32 // CHUNK, chunk, 0)
    plsc.subcore_barrier()
    pltpu.sync_copy(deg_sh.at[pl.ds(t0, NPAD // 16)],
                    deg_out.at[pl.ds(c * NPAD + t0, NPAD // 16)])


@functools.partial(
    pl.kernel,
    out_type=jax.ShapeDtypeStruct((NPAD, D), jnp.float32),
    mesh=_mesh,
    scratch_types=[
        pltpu.VMEM((CHUNK,), jnp.int32),
        pltpu.VMEM((8, 128), jnp.int32),
        pltpu.VMEM((8, 128), jnp.int32),
        pltpu.VMEM((CHUNK // 4, D), jnp.float32),
        pltpu.VMEM((CHUNK // 4, D), jnp.float32),
        pltpu.VMEM_SHARED((HALF + 16, D), jnp.float32),
        pltpu.SemaphoreType.DMA,
        pltpu.SemaphoreType.DMA,
    ],
)
def _prop_kernel(hs_hbm, rows_hbm, col2d_hbm, acc_out,
                 rbuf, cbuf, lbuf, gbuf_a, gbuf_b, acc_sh, sem_a, sem_b):
    c = lax.axis_index("c")
    s = lax.axis_index("s")
    t0 = s * RPT
    pltpu.sync_copy(hs_hbm.at[pl.ds(c * HALF + t0, RPT)],
                    acc_sh.at[pl.ds(t0, RPT)])
    plsc.subcore_barrier()

    lo = c * HALF
    dummy = HALF + s
    ebase = s * (EPAD // 16)

    Q = CHUNK // 4
    bufs = (gbuf_a, gbuf_b)
    sems = (sem_a, sem_b)

    def chunk(j, carry):
        b = ebase + j * CHUNK
        pltpu.sync_copy(rows_hbm.at[pl.ds(b, CHUNK)], rbuf)
        cb = pl.multiple_of(b // 128, 8)
        pltpu.sync_copy(col2d_hbm.at[pl.ds(cb, 8)], cbuf)
        for k in range(8):
            for g in range(D // 16):
                v = cbuf[k, pl.ds(g * 16, 16)]
                lc = v - lo
                ok = (lc >= 0) & (lc < HALF)
                lbuf[k, pl.ds(g * 16, 16)] = jnp.where(ok, lc, dummy)
        descs = [None] * 4
        descs[0] = pltpu.async_copy(
            hs_hbm.at[rbuf.at[pl.ds(0, Q)]], bufs[0], sems[0])
        for q in range(4):
            if q < 3:
                descs[q + 1] = pltpu.async_copy(
                    hs_hbm.at[rbuf.at[pl.ds((q + 1) * Q, Q)]],
                    bufs[(q + 1) % 2], sems[(q + 1) % 2])
            descs[q].wait()
            for k in range(2):
                pltpu.sync_copy(bufs[q % 2].at[pl.ds(k * 128, 128)],
                                acc_sh.at[lbuf.at[q * 2 + k]], add=True)
        return carry

    lax.fori_loop(0, NCHUNK, chunk, 0)
    plsc.subcore_barrier()
    pltpu.sync_copy(acc_sh.at[pl.ds(t0, RPT)],
                    acc_out.at[pl.ds(c * HALF + t0, RPT)])


def _pre_body(x_ref, w_ref, deg_ref, out_ref):
    s = lax.rsqrt(deg_ref[...])[:, None]
    h = jnp.dot(x_ref[...], w_ref[...].T, preferred_element_type=jnp.float32)
    out_ref[...] = h * s


def _pre_call(x, W1, deg):
    return pl.pallas_call(
        _pre_body,
        grid=(NPAD // 1024,),
        in_specs=[
            pl.BlockSpec((1024, D), lambda i: (i, 0)),
            pl.BlockSpec((D, D), lambda i: (0, 0)),
            pl.BlockSpec((1024,), lambda i: (i,)),
        ],
        out_specs=pl.BlockSpec((1024, D), lambda i: (i, 0)),
        out_shape=jax.ShapeDtypeStruct((NPAD, D), jnp.float32),
    )(x, W1, deg)


def _mid_body(acc_ref, deg_ref, b_ref, w_ref, out_ref):
    s = lax.rsqrt(deg_ref[...])[:, None]
    x1 = acc_ref[...] * s + b_ref[...][None, :]
    xr = jnp.maximum(x1, 0.0)
    h2 = jnp.dot(xr, w_ref[...].T, preferred_element_type=jnp.float32)
    out_ref[...] = h2 * s


def _mid_call(acc, deg, b1, W2):
    return pl.pallas_call(
        _mid_body,
        grid=(NPAD // 1024,),
        in_specs=[
            pl.BlockSpec((1024, D), lambda i: (i, 0)),
            pl.BlockSpec((1024,), lambda i: (i,)),
            pl.BlockSpec((D,), lambda i: (0,)),
            pl.BlockSpec((D, D), lambda i: (0, 0)),
        ],
        out_specs=pl.BlockSpec((1024, D), lambda i: (i, 0)),
        out_shape=jax.ShapeDtypeStruct((NPAD, D), jnp.float32),
    )(acc, deg, b1, W2)


def _final_body(acc_ref, deg_ref, b_ref, out_ref):
    s = lax.rsqrt(deg_ref[...])[:, None]
    out_ref[...] = acc_ref[...] * s + b_ref[...][None, :]


def _final_call(acc, deg, b2):
    return pl.pallas_call(
        _final_body,
        grid=(NPAD // 1024,),
        in_specs=[
            pl.BlockSpec((1024, D), lambda i: (i, 0)),
            pl.BlockSpec((1024,), lambda i: (i,)),
            pl.BlockSpec((D,), lambda i: (0,)),
        ],
        out_specs=pl.BlockSpec((1024, D), lambda i: (i, 0)),
        out_shape=jax.ShapeDtypeStruct((NPAD, D), jnp.float32),
    )(acc, deg, b2)


def kernel(x, edge_index, W1, b1, W2, b2):
    ei = edge_index.astype(jnp.int32)
    row, col = ei[0], ei[1]
    pad = EPAD - E
    row_p = jnp.concatenate([row, jnp.zeros((pad,), jnp.int32)])
    col_p = jnp.concatenate([col, jnp.full((pad,), N, jnp.int32)])
    col2d = col_p.reshape(EPAD // 128, 128)

    deg2 = _deg_kernel(col2d, jnp.asarray(_ONES))
    deg = deg2[:NPAD, 0] + deg2[NPAD:, 0] + 1.0

    hs1 = _pre_call(x, W1, deg)
    acc1 = _prop_kernel(hs1, row_p, col2d)
    hs2 = _mid_call(acc1, deg, b1, W2)
    acc2 = _prop_kernel(hs2, row_p, col2d)
    return _final_call(acc2, deg, b2)[:N]


# speedup vs baseline: 6.0973x; 1.0420x over previous
import functools

import jax
import jax.numpy as jnp
import numpy as np
from jax import lax
from jax.experimental import pallas as pl
from jax.experimental.pallas import tpu as pltpu
from jax.experimental.pallas import tpu_sc as plsc

N = 10000
E = 320000
D = 128
NPAD = 10240
HALF = NPAD // 2
RPT = HALF // 16
EPAD = 327680
CHUNK = 1024
NCHUNK = EPAD // (16 * CHUNK)

_mesh = plsc.VectorSubcoreMesh(core_axis_name="c", subcore_axis_name="s")

_ONES = np.ones((128, D), np.float32)


@functools.partial(
    pl.kernel,
    out_type=jax.ShapeDtypeStruct((2 * NPAD, D), jnp.float32),
    mesh=_mesh,
    scratch_types=[
        pltpu.VMEM((8, 128), jnp.int32),
        pltpu.VMEM((128, D), jnp.float32),
        pltpu.VMEM((16, D), jnp.float32),
        pltpu.VMEM_SHARED((NPAD, D), jnp.float32),
    ],
)
def _deg_kernel(col2d_hbm, ones_hbm, deg_out, cbuf, ones_v, zbuf, deg_sh):
    c = lax.axis_index("c")
    s = lax.axis_index("s")
    t0 = s * (NPAD // 16)
    pltpu.sync_copy(ones_hbm, ones_v)
    zero16 = jnp.zeros((16,), jnp.float32)
    for r in range(16):
        for g in range(D // 16):
            zbuf[r, pl.ds(g * 16, 16)] = zero16
    for r in range(NPAD // 16 // 16):
        pltpu.sync_copy(zbuf, deg_sh.at[pl.ds(t0 + r * 16, 16)])
    plsc.subcore_barrier()

    rowbase0 = c * (EPAD // 2 // 128) + s * (EPAD // 32 // 128)

    def chunk(j, carry):
        cb = pl.multiple_of(rowbase0 + j * 8, 8)
        pltpu.sync_copy(col2d_hbm.at[pl.ds(cb, 8)], cbuf)
        for k in range(8):
            pltpu.sync_copy(ones_v, deg_sh.at[cbuf.at[k]], add=True)
        return carry

    lax.fori_loop(0, EPAD // 32 // CHUNK, chunk, 0)
    plsc.subcore_barrier()
    pltpu.sync_copy(deg_sh.at[pl.ds(t0, NPAD // 16)],
                    deg_out.at[pl.ds(c * NPAD + t0, NPAD // 16)])


EPT = EPAD // 16
Q = 128
NSUB = EPT // Q


@functools.partial(
    pl.kernel,
    out_type=jax.ShapeDtypeStruct((NPAD, D), jnp.float32),
    mesh=_mesh,
    scratch_types=[
        pltpu.VMEM((EPT,), jnp.int32),
        pltpu.VMEM((EPT // 128, 128), jnp.int32),
        pltpu.VMEM((Q, D), jnp.float32),
        pltpu.VMEM((Q, D), jnp.float32),
        pltpu.VMEM_SHARED((HALF + 16, D), jnp.float32),
        pltpu.SemaphoreType.DMA,
        pltpu.SemaphoreType.DMA,
    ],
)
def _prop_kernel(hs_hbm, rows_hbm, coll_hbm, acc_out,
                 rbig, cbig, gbuf_a, gbuf_b, acc_sh, sem_a, sem_b):
    c = lax.axis_index("c")
    s = lax.axis_index("s")
    t0 = s * RPT
    pltpu.sync_copy(hs_hbm.at[pl.ds(c * HALF + t0, RPT)],
                    acc_sh.at[pl.ds(t0, RPT)])
    plsc.subcore_barrier()

    pltpu.sync_copy(rows_hbm.at[pl.ds(s * EPT, EPT)], rbig)
    crow = pl.multiple_of(c * (EPAD // 128) + s * (EPT // 128), 8)
    pltpu.sync_copy(coll_hbm.at[pl.ds(crow, EPT // 128)], cbig)

    bufs = (gbuf_a, gbuf_b)
    sems = (sem_a, sem_b)

    def gather(q, p):
        pltpu.async_copy(
            hs_hbm.at[rbig.at[pl.ds(q * Q, Q)]], bufs[p], sems[p])

    def drain(p):
        pltpu.make_async_copy(hs_hbm.at[pl.ds(0, Q)], bufs[p], sems[p]).wait()

    def scatter(q, p):
        pltpu.sync_copy(bufs[p], acc_sh.at[cbig.at[q]], add=True)

    gather(0, 0)

    def body(i, carry):
        q0 = 2 * i
        gather(q0 + 1, 1)
        drain(0)
        scatter(q0, 0)
        gather(q0 + 2, 0)
        drain(1)
        scatter(q0 + 1, 1)
        return carry

    lax.fori_loop(0, NSUB // 2 - 1, body, 0)
    gather(NSUB - 1, 1)
    drain(0)
    scatter(NSUB - 2, 0)
    drain(1)
    scatter(NSUB - 1, 1)
    plsc.subcore_barrier()
    pltpu.sync_copy(acc_sh.at[pl.ds(t0, RPT)],
                    acc_out.at[pl.ds(c * HALF + t0, RPT)])


def _pre_body(x_ref, w_ref, deg_ref, out_ref):
    s = lax.rsqrt(deg_ref[...])[:, None]
    h = jnp.dot(x_ref[...], w_ref[...].T, preferred_element_type=jnp.float32)
    out_ref[...] = h * s


def _pre_call(x, W1, deg):
    return pl.pallas_call(
        _pre_body,
        grid=(NPAD // 1024,),
        in_specs=[
            pl.BlockSpec((1024, D), lambda i: (i, 0)),
            pl.BlockSpec((D, D), lambda i: (0, 0)),
            pl.BlockSpec((1024,), lambda i: (i,)),
        ],
        out_specs=pl.BlockSpec((1024, D), lambda i: (i, 0)),
        out_shape=jax.ShapeDtypeStruct((NPAD, D), jnp.float32),
    )(x, W1, deg)


def _mid_body(acc_ref, deg_ref, b_ref, w_ref, out_ref):
    s = lax.rsqrt(deg_ref[...])[:, None]
    x1 = acc_ref[...] * s + b_ref[...][None, :]
    xr = jnp.maximum(x1, 0.0)
    h2 = jnp.dot(xr, w_ref[...].T, preferred_element_type=jnp.float32)
    out_ref[...] = h2 * s


def _mid_call(acc, deg, b1, W2):
    return pl.pallas_call(
        _mid_body,
        grid=(NPAD // 1024,),
        in_specs=[
            pl.BlockSpec((1024, D), lambda i: (i, 0)),
            pl.BlockSpec((1024,), lambda i: (i,)),
            pl.BlockSpec((D,), lambda i: (0,)),
            pl.BlockSpec((D, D), lambda i: (0, 0)),
        ],
        out_specs=pl.BlockSpec((1024, D), lambda i: (i, 0)),
        out_shape=jax.ShapeDtypeStruct((NPAD, D), jnp.float32),
    )(acc, deg, b1, W2)


def _final_body(acc_ref, deg_ref, b_ref, out_ref):
    s = lax.rsqrt(deg_ref[...])[:, None]
    out_ref[...] = acc_ref[...] * s + b_ref[...][None, :]


def _final_call(acc, deg, b2):
    return pl.pallas_call(
        _final_body,
        grid=(NPAD // 1024,),
        in_specs=[
            pl.BlockSpec((1024, D), lambda i: (i, 0)),
            pl.BlockSpec((1024,), lambda i: (i,)),
            pl.BlockSpec((D,), lambda i: (0,)),
        ],
        out_specs=pl.BlockSpec((1024, D), lambda i: (i, 0)),
        out_shape=jax.ShapeDtypeStruct((NPAD, D), jnp.float32),
    )(acc, deg, b2)


def kernel(x, edge_index, W1, b1, W2, b2):
    ei = edge_index.astype(jnp.int32)
    row, col = ei[0], ei[1]
    pad = EPAD - E
    row_p = jnp.concatenate([row, jnp.zeros((pad,), jnp.int32)])
    col_p = jnp.concatenate([col, jnp.full((pad,), N, jnp.int32)])
    col2d = col_p.reshape(EPAD // 128, 128)

    tile_of_e = (jnp.arange(EPAD, dtype=jnp.int32) // EPT) % 16
    halves = []
    for cc in (0, 1):
        lc = col_p - cc * HALF
        okc = (lc >= 0) & (lc < HALF)
        halves.append(jnp.where(okc, lc, HALF + tile_of_e))
    coll = jnp.concatenate(halves).reshape(2 * EPAD // 128, 128)

    deg2 = _deg_kernel(col2d, jnp.asarray(_ONES))
    deg = deg2[:NPAD, 0] + deg2[NPAD:, 0] + 1.0

    hs1 = _pre_call(x, W1, deg)
    acc1 = _prop_kernel(hs1, row_p, coll)
    hs2 = _mid_call(acc1, deg, b1, W2)
    acc2 = _prop_kernel(hs2, row_p, coll)
    return _final_call(acc2, deg, b2)[:N]


# speedup vs baseline: 13.3428x; 2.1883x over previous
import functools

import jax
import jax.numpy as jnp
import numpy as np
from jax import lax
from jax.experimental import pallas as pl
from jax.experimental.pallas import tpu as pltpu
from jax.experimental.pallas import tpu_sc as plsc

N = 10000
E = 320000
D = 128
NPAD = 10240
HALF = NPAD // 2
RPT = HALF // 16
EPAD = 327680
CHUNK = 1024
NCHUNK = EPAD // (16 * CHUNK)

_mesh = plsc.VectorSubcoreMesh(core_axis_name="c", subcore_axis_name="s")

_ONES = np.ones((128, D), np.float32)


@functools.partial(
    pl.kernel,
    out_type=jax.ShapeDtypeStruct((2 * NPAD, D), jnp.float32),
    mesh=_mesh,
    scratch_types=[
        pltpu.VMEM((8, 128), jnp.int32),
        pltpu.VMEM((128, D), jnp.float32),
        pltpu.VMEM((16, D), jnp.float32),
        pltpu.VMEM_SHARED((NPAD, D), jnp.float32),
    ],
)
def _deg_kernel(col2d_hbm, ones_hbm, deg_out, cbuf, ones_v, zbuf, deg_sh):
    c = lax.axis_index("c")
    s = lax.axis_index("s")
    t0 = s * (NPAD // 16)
    pltpu.sync_copy(ones_hbm, ones_v)
    zero16 = jnp.zeros((16,), jnp.float32)
    for r in range(16):
        for g in range(D // 16):
            zbuf[r, pl.ds(g * 16, 16)] = zero16
    for r in range(NPAD // 16 // 16):
        pltpu.sync_copy(zbuf, deg_sh.at[pl.ds(t0 + r * 16, 16)])
    plsc.subcore_barrier()

    rowbase0 = c * (EPAD // 2 // 128) + s * (EPAD // 32 // 128)

    def chunk(j, carry):
        cb = pl.multiple_of(rowbase0 + j * 8, 8)
        pltpu.sync_copy(col2d_hbm.at[pl.ds(cb, 8)], cbuf)
        for k in range(8):
            pltpu.sync_copy(ones_v, deg_sh.at[cbuf.at[k]], add=True)
        return carry

    lax.fori_loop(0, EPAD // 32 // CHUNK, chunk, 0)
    plsc.subcore_barrier()
    pltpu.sync_copy(deg_sh.at[pl.ds(t0, NPAD // 16)],
                    deg_out.at[pl.ds(c * NPAD + t0, NPAD // 16)])


H = D // 2
EPT = EPAD // 16
Q = 256
NSUB = EPT // Q


@functools.partial(
    pl.kernel,
    out_type=jax.ShapeDtypeStruct((2 * NPAD, H), jnp.float32),
    mesh=_mesh,
    scratch_types=[
        pltpu.VMEM((EPT,), jnp.int32),
        pltpu.VMEM((EPT // 128, 128), jnp.int32),
        pltpu.VMEM((Q, H), jnp.float32),
        pltpu.VMEM((Q, H), jnp.float32),
        pltpu.VMEM_SHARED((NPAD + 16, H), jnp.float32),
        pltpu.SemaphoreType.DMA,
        pltpu.SemaphoreType.DMA,
    ],
    compiler_params=pltpu.CompilerParams(use_tc_tiling_on_sc=False),
)
def _prop_kernel(hs_hbm, rows2_hbm, col2d_hbm, acc_out,
                 rbig, cbig, gbuf_a, gbuf_b, acc_sh, sem_a, sem_b):
    c = lax.axis_index("c")
    s = lax.axis_index("s")
    t0 = s * (NPAD // 16)
    pltpu.sync_copy(hs_hbm.at[pl.ds(c * NPAD + t0, NPAD // 16)],
                    acc_sh.at[pl.ds(t0, NPAD // 16)])
    plsc.subcore_barrier()

    pltpu.sync_copy(rows2_hbm.at[pl.ds(c * EPAD + s * EPT, EPT)], rbig)
    crow = pl.multiple_of(s * (EPT // 128), 8)
    pltpu.sync_copy(col2d_hbm.at[pl.ds(crow, EPT // 128)], cbig)

    bufs = (gbuf_a, gbuf_b)
    sems = (sem_a, sem_b)

    def gather(q, p):
        pltpu.async_copy(
            hs_hbm.at[rbig.at[pl.ds(q * Q, Q)]], bufs[p], sems[p])

    def drain(p):
        pltpu.make_async_copy(hs_hbm.at[pl.ds(0, Q)], bufs[p], sems[p]).wait()

    def scatter(q, p):
        for k in range(2):
            pltpu.sync_copy(bufs[p].at[pl.ds(k * 128, 128)],
                            acc_sh.at[cbig.at[2 * q + k]], add=True)

    gather(0, 0)

    def body(i, carry):
        q0 = 2 * i
        gather(q0 + 1, 1)
        drain(0)
        scatter(q0, 0)
        gather(q0 + 2, 0)
        drain(1)
        scatter(q0 + 1, 1)
        return carry

    lax.fori_loop(0, NSUB // 2 - 1, body, 0)
    gather(NSUB - 1, 1)
    drain(0)
    scatter(NSUB - 2, 0)
    drain(1)
    scatter(NSUB - 1, 1)
    plsc.subcore_barrier()
    pltpu.sync_copy(acc_sh.at[pl.ds(t0, NPAD // 16)],
                    acc_out.at[pl.ds(c * NPAD + t0, NPAD // 16)])


def _pre_body(x_ref, w_ref, deg_ref, out_ref):
    s = lax.rsqrt(deg_ref[...])[:, None]
    h = jnp.dot(x_ref[...], w_ref[...].T, preferred_element_type=jnp.float32)
    hs = h * s
    out_ref[0] = hs[:, :H]
    out_ref[1] = hs[:, H:]


def _pre_call(x, W1, deg):
    return pl.pallas_call(
        _pre_body,
        grid=(NPAD // 1024,),
        in_specs=[
            pl.BlockSpec((1024, D), lambda i: (i, 0)),
            pl.BlockSpec((D, D), lambda i: (0, 0)),
            pl.BlockSpec((1024,), lambda i: (i,)),
        ],
        out_specs=pl.BlockSpec((2, 1024, H), lambda i: (0, i, 0)),
        out_shape=jax.ShapeDtypeStruct((2, NPAD, H), jnp.float32),
    )(x, W1, deg)


def _mid_body(acc_ref, deg_ref, b_ref, w_ref, out_ref):
    s = lax.rsqrt(deg_ref[...])[:, None]
    x1 = (jnp.concatenate([acc_ref[0], acc_ref[1]], axis=1) * s
          + b_ref[...][None, :])
    xr = jnp.maximum(x1, 0.0)
    h2 = jnp.dot(xr, w_ref[...].T, preferred_element_type=jnp.float32)
    hs2 = h2 * s
    out_ref[0] = hs2[:, :H]
    out_ref[1] = hs2[:, H:]


def _mid_call(acc, deg, b1, W2):
    return pl.pallas_call(
        _mid_body,
        grid=(NPAD // 1024,),
        in_specs=[
            pl.BlockSpec((2, 1024, H), lambda i: (0, i, 0)),
            pl.BlockSpec((1024,), lambda i: (i,)),
            pl.BlockSpec((D,), lambda i: (0,)),
            pl.BlockSpec((D, D), lambda i: (0, 0)),
        ],
        out_specs=pl.BlockSpec((2, 1024, H), lambda i: (0, i, 0)),
        out_shape=jax.ShapeDtypeStruct((2, NPAD, H), jnp.float32),
    )(acc, deg, b1, W2)


def _final_body(acc_ref, deg_ref, b_ref, out_ref):
    s = lax.rsqrt(deg_ref[...])[:, None]
    out_ref[...] = (jnp.concatenate([acc_ref[0], acc_ref[1]], axis=1) * s
                    + b_ref[...][None, :])


def _final_call(acc, deg, b2):
    return pl.pallas_call(
        _final_body,
        grid=(NPAD // 1024,),
        in_specs=[
            pl.BlockSpec((2, 1024, H), lambda i: (0, i, 0)),
            pl.BlockSpec((1024,), lambda i: (i,)),
            pl.BlockSpec((D,), lambda i: (0,)),
        ],
        out_specs=pl.BlockSpec((1024, D), lambda i: (i, 0)),
        out_shape=jax.ShapeDtypeStruct((NPAD, D), jnp.float32),
    )(acc, deg, b2)


def kernel(x, edge_index, W1, b1, W2, b2):
    ei = edge_index.astype(jnp.int32)
    row, col = ei[0], ei[1]
    pad = EPAD - E
    row_p = jnp.concatenate([row, jnp.zeros((pad,), jnp.int32)])
    col_p = jnp.concatenate([col, jnp.full((pad,), N, jnp.int32)])
    col2d = col_p.reshape(EPAD // 128, 128)

    rows2 = jnp.concatenate([row_p, row_p + NPAD])

    deg2 = _deg_kernel(col2d, jnp.asarray(_ONES))
    deg = deg2[:NPAD, 0] + deg2[NPAD:, 0] + 1.0

    hs1 = _pre_call(x, W1, deg)
    acc1 = _prop_kernel(hs1.reshape(2 * NPAD, H), rows2, col2d)
    hs2 = _mid_call(acc1.reshape(2, NPAD, H), deg, b1, W2)
    acc2 = _prop_kernel(hs2.reshape(2 * NPAD, H), rows2, col2d)
    return _final_call(acc2.reshape(2, NPAD, H), deg, b2)[:N]


# speedup vs baseline: 14.2246x; 1.0661x over previous
import functools

import jax
import jax.numpy as jnp
import numpy as np
from jax import lax
from jax.experimental import pallas as pl
from jax.experimental.pallas import tpu as pltpu
from jax.experimental.pallas import tpu_sc as plsc

N = 10000
E = 320000
D = 128
NPAD = 10240
HALF = NPAD // 2
RPT = HALF // 16
EPAD = 327680
CHUNK = 1024
NCHUNK = EPAD // (16 * CHUNK)

_mesh = plsc.VectorSubcoreMesh(core_axis_name="c", subcore_axis_name="s")

DW = 16
EPT32 = EPAD // 32
_ONES = np.ones((128, DW), np.float32)
_ZROW = np.zeros((NPAD // 16 + 1, DW), np.float32)


@functools.partial(
    pl.kernel,
    out_type=jax.ShapeDtypeStruct((2 * NPAD, DW), jnp.float32),
    mesh=_mesh,
    scratch_types=[
        pltpu.VMEM((EPT32 // 128, 128), jnp.int32),
        pltpu.VMEM((128, DW), jnp.float32),
        pltpu.VMEM_SHARED((NPAD + 16, DW), jnp.float32),
    ],
    compiler_params=pltpu.CompilerParams(use_tc_tiling_on_sc=False),
)
def _deg_kernel(col2d_hbm, ones_hbm, zeros_hbm, deg_out, cbig, ones_v, deg_sh):
    c = lax.axis_index("c")
    s = lax.axis_index("s")
    t0 = s * (NPAD // 16)
    pltpu.sync_copy(ones_hbm, ones_v)
    pltpu.sync_copy(zeros_hbm.at[pl.ds(0, NPAD // 16)],
                    deg_sh.at[pl.ds(t0, NPAD // 16)])
    crow = pl.multiple_of(c * (EPAD // 2 // 128) + s * (EPT32 // 128), 8)
    pltpu.sync_copy(col2d_hbm.at[pl.ds(crow, EPT32 // 128)], cbig)
    plsc.subcore_barrier()

    def chunk(j, carry):
        pltpu.sync_copy(ones_v, deg_sh.at[cbig.at[j]], add=True)
        return carry

    lax.fori_loop(0, EPT32 // 128, chunk, 0)
    plsc.subcore_barrier()
    pltpu.sync_copy(deg_sh.at[pl.ds(t0, NPAD // 16)],
                    deg_out.at[pl.ds(c * NPAD + t0, NPAD // 16)])


H = D // 2
EPT = EPAD // 16
Q = 256
NSUB = EPT // Q


@functools.partial(
    pl.kernel,
    out_type=jax.ShapeDtypeStruct((2 * NPAD, H), jnp.float32),
    mesh=_mesh,
    scratch_types=[
        pltpu.VMEM((EPT,), jnp.int32),
        pltpu.VMEM((EPT // 128, 128), jnp.int32),
        pltpu.VMEM((Q, H), jnp.float32),
        pltpu.VMEM((Q, H), jnp.float32),
        pltpu.VMEM_SHARED((NPAD + 16, H), jnp.float32),
        pltpu.SemaphoreType.DMA,
        pltpu.SemaphoreType.DMA,
    ],
    compiler_params=pltpu.CompilerParams(use_tc_tiling_on_sc=False),
)
def _prop_kernel(hs_hbm, rows2_hbm, col2d_hbm, acc_out,
                 rbig, cbig, gbuf_a, gbuf_b, acc_sh, sem_a, sem_b):
    c = lax.axis_index("c")
    s = lax.axis_index("s")
    t0 = s * (NPAD // 16)
    pltpu.sync_copy(hs_hbm.at[pl.ds(c * NPAD + t0, NPAD // 16)],
                    acc_sh.at[pl.ds(t0, NPAD // 16)])
    plsc.subcore_barrier()

    pltpu.sync_copy(rows2_hbm.at[pl.ds(c * EPAD + s * EPT, EPT)], rbig)
    crow = pl.multiple_of(s * (EPT // 128), 8)
    pltpu.sync_copy(col2d_hbm.at[pl.ds(crow, EPT // 128)], cbig)

    bufs = (gbuf_a, gbuf_b)
    sems = (sem_a, sem_b)

    def gather(q, p):
        pltpu.async_copy(
            hs_hbm.at[rbig.at[pl.ds(q * Q, Q)]], bufs[p], sems[p])

    def drain(p):
        pltpu.make_async_copy(hs_hbm.at[pl.ds(0, Q)], bufs[p], sems[p]).wait()

    def scatter(q, p):
        for k in range(2):
            pltpu.sync_copy(bufs[p].at[pl.ds(k * 128, 128)],
                            acc_sh.at[cbig.at[2 * q + k]], add=True)

    gather(0, 0)

    def body(i, carry):
        q0 = 2 * i
        gather(q0 + 1, 1)
        drain(0)
        scatter(q0, 0)
        gather(q0 + 2, 0)
        drain(1)
        scatter(q0 + 1, 1)
        return carry

    lax.fori_loop(0, NSUB // 2 - 1, body, 0)
    gather(NSUB - 1, 1)
    drain(0)
    scatter(NSUB - 2, 0)
    drain(1)
    scatter(NSUB - 1, 1)
    plsc.subcore_barrier()
    pltpu.sync_copy(acc_sh.at[pl.ds(t0, NPAD // 16)],
                    acc_out.at[pl.ds(c * NPAD + t0, NPAD // 16)])


def _pre_body(x_ref, w_ref, deg_ref, out_ref):
    s = lax.rsqrt(deg_ref[...])[:, None]
    h = jnp.dot(x_ref[...], w_ref[...].T, preferred_element_type=jnp.float32)
    hs = h * s
    out_ref[0] = hs[:, :H]
    out_ref[1] = hs[:, H:]


def _pre_call(x, W1, deg):
    return pl.pallas_call(
        _pre_body,
        grid=(NPAD // 1024,),
        in_specs=[
            pl.BlockSpec((1024, D), lambda i: (i, 0)),
            pl.BlockSpec((D, D), lambda i: (0, 0)),
            pl.BlockSpec((1024,), lambda i: (i,)),
        ],
        out_specs=pl.BlockSpec((2, 1024, H), lambda i: (0, i, 0)),
        out_shape=jax.ShapeDtypeStruct((2, NPAD, H), jnp.float32),
    )(x, W1, deg)


def _mid_body(acc_ref, deg_ref, b_ref, w_ref, out_ref):
    s = lax.rsqrt(deg_ref[...])[:, None]
    x1 = (jnp.concatenate([acc_ref[0], acc_ref[1]], axis=1) * s
          + b_ref[...][None, :])
    xr = jnp.maximum(x1, 0.0)
    h2 = jnp.dot(xr, w_ref[...].T, preferred_element_type=jnp.float32)
    hs2 = h2 * s
    out_ref[0] = hs2[:, :H]
    out_ref[1] = hs2[:, H:]


def _mid_call(acc, deg, b1, W2):
    return pl.pallas_call(
        _mid_body,
        grid=(NPAD // 1024,),
        in_specs=[
            pl.BlockSpec((2, 1024, H), lambda i: (0, i, 0)),
            pl.BlockSpec((1024,), lambda i: (i,)),
            pl.BlockSpec((D,), lambda i: (0,)),
            pl.BlockSpec((D, D), lambda i: (0, 0)),
        ],
        out_specs=pl.BlockSpec((2, 1024, H), lambda i: (0, i, 0)),
        out_shape=jax.ShapeDtypeStruct((2, NPAD, H), jnp.float32),
    )(acc, deg, b1, W2)


def _final_body(acc_ref, deg_ref, b_ref, out_ref):
    s = lax.rsqrt(deg_ref[...])[:, None]
    out_ref[...] = (jnp.concatenate([acc_ref[0], acc_ref[1]], axis=1) * s
                    + b_ref[...][None, :])


def _final_call(acc, deg, b2):
    return pl.pallas_call(
        _final_body,
        grid=(NPAD // 1024,),
        in_specs=[
            pl.BlockSpec((2, 1024, H), lambda i: (0, i, 0)),
            pl.BlockSpec((1024,), lambda i: (i,)),
            pl.BlockSpec((D,), lambda i: (0,)),
        ],
        out_specs=pl.BlockSpec((1024, D), lambda i: (i, 0)),
        out_shape=jax.ShapeDtypeStruct((NPAD, D), jnp.float32),
    )(acc, deg, b2)


def kernel(x, edge_index, W1, b1, W2, b2):
    ei = edge_index.astype(jnp.int32)
    row, col = ei[0], ei[1]
    pad = EPAD - E
    row_p = jnp.concatenate([row, jnp.zeros((pad,), jnp.int32)])
    col_p = jnp.concatenate([col, jnp.full((pad,), N, jnp.int32)])
    col2d = col_p.reshape(EPAD // 128, 128)

    rows2 = jnp.concatenate([row_p, row_p + NPAD])

    deg2 = _deg_kernel(col2d, jnp.asarray(_ONES), jnp.asarray(_ZROW))
    deg = deg2[:NPAD, 0] + deg2[NPAD:, 0] + 1.0

    hs1 = _pre_call(x, W1, deg)
    acc1 = _prop_kernel(hs1.reshape(2 * NPAD, H), rows2, col2d)
    hs2 = _mid_call(acc1.reshape(2, NPAD, H), deg, b1, W2)
    acc2 = _prop_kernel(hs2.reshape(2 * NPAD, H), rows2, col2d)
    return _final_call(acc2.reshape(2, NPAD, H), deg, b2)[:N]
